# initial kernel scaffold (unmeasured)
import jax
import jax.numpy as jnp
from jax import lax
from jax.experimental import pallas as pl
from jax.experimental.pallas import tpu as pltpu


def kernel(
    x,
):
    def body(*refs):
        pass

    out_shape = jax.ShapeDtypeStruct(..., jnp.float32)
    return pl.pallas_call(body, out_shape=out_shape)(...)


# baseline (device time: 25685 ns/iter reference)
import jax
import jax.numpy as jnp
from jax import lax
from jax.experimental import pallas as pl
from jax.experimental.pallas import tpu as pltpu

K = 16
NEG = float("-inf")


def kernel(x):
    m, n = x.shape

    def body(x_ref, out_ref, send_ref, recv_ref, send_sem, recv_sem):
        my_x = lax.axis_index("x")
        my_y = lax.axis_index("y")
        my_z = lax.axis_index("z")
        nbr = (my_x, 1 - my_y, my_z)

        barrier_sem = pltpu.get_barrier_semaphore()
        pl.semaphore_signal(
            barrier_sem, inc=1, device_id=nbr,
            device_id_type=pl.DeviceIdType.MESH,
        )
        pl.semaphore_wait(barrier_sem, 1)

        kcol = lax.broadcasted_iota(jnp.int32, (m, K), 1)

        xw = x_ref[:, :]
        vals = jnp.full((m, K), NEG, jnp.float32)
        for k in range(K):
            cur = jnp.max(xw, axis=1, keepdims=True)
            vals = jnp.where(kcol == k, cur, vals)
            xw = jnp.where(xw == cur, NEG, xw)
        send_ref[:, :] = vals

        rdma = pltpu.make_async_remote_copy(
            src_ref=send_ref,
            dst_ref=recv_ref,
            send_sem=send_sem,
            recv_sem=recv_sem,
            device_id=nbr,
            device_id_type=pl.DeviceIdType.MESH,
        )
        rdma.start()
        rdma.wait()

        cand = jnp.concatenate([vals, recv_ref[:, :]], axis=1)
        out = jnp.full((m, K), NEG, jnp.float32)
        for k in range(K):
            cur = jnp.max(cand, axis=1, keepdims=True)
            out = jnp.where(kcol == k, cur, out)
            cand = jnp.where(cand == cur, NEG, cand)
        out_ref[:, :] = out

    return pl.pallas_call(
        body,
        out_shape=jax.ShapeDtypeStruct((m, K), jnp.float32),
        in_specs=[pl.BlockSpec(memory_space=pltpu.VMEM)],
        out_specs=pl.BlockSpec(memory_space=pltpu.VMEM),
        scratch_shapes=[
            pltpu.VMEM((m, K), jnp.float32),
            pltpu.VMEM((m, K), jnp.float32),
            pltpu.SemaphoreType.DMA,
            pltpu.SemaphoreType.DMA,
        ],
        compiler_params=pltpu.CompilerParams(collective_id=0),
    )(x)


# device time: 21397 ns/iter; 1.2004x vs baseline; 1.2004x over previous
import jax
import jax.numpy as jnp
from jax import lax
from jax.experimental import pallas as pl
from jax.experimental.pallas import tpu as pltpu

K = 16
NEG = float("-inf")
FOLD_LEVELS = 4
N_CAND = 81


def kernel(x):
    m, n = x.shape

    def extract_topk(a, k):
        if k == 1:
            return jnp.max(a, axis=1, keepdims=True)
        kcol = lax.broadcasted_iota(jnp.int32, (m, k), 1)
        vals = jnp.full((m, k), NEG, jnp.float32)
        for r in range(k):
            cur = jnp.max(a, axis=1, keepdims=True)
            vals = jnp.where(kcol == r, cur, vals)
            if r < k - 1:
                a = jnp.where(a == cur, NEG, a)
        return vals

    def body(x_ref, out_ref, send_ref, recv_ref, send_sem, recv_sem):
        my_x = lax.axis_index("x")
        my_y = lax.axis_index("y")
        my_z = lax.axis_index("z")
        nbr = (my_x, 1 - my_y, my_z)

        jobs = [(x_ref[:, :], K)]
        for _ in range(FOLD_LEVELS):
            nxt = []
            for a, k in jobs:
                h = a.shape[1] // 2
                lo, hi = a[:, :h], a[:, h:]
                nxt.append((jnp.maximum(lo, hi), k))
                if k >= 2:
                    nxt.append((jnp.minimum(lo, hi), k // 2))
            jobs = nxt

        cands = jnp.concatenate(
            [extract_topk(a, k) for a, k in jobs], axis=1
        )
        send_ref[:, :] = cands

        barrier_sem = pltpu.get_barrier_semaphore()
        pl.semaphore_signal(
            barrier_sem, inc=1, device_id=nbr,
            device_id_type=pl.DeviceIdType.MESH,
        )
        pl.semaphore_wait(barrier_sem, 1)

        rdma = pltpu.make_async_remote_copy(
            src_ref=send_ref,
            dst_ref=recv_ref,
            send_sem=send_sem,
            recv_sem=recv_sem,
            device_id=nbr,
            device_id_type=pl.DeviceIdType.MESH,
        )
        rdma.start()
        rdma.wait()

        allc = jnp.concatenate([cands, recv_ref[:, :]], axis=1)
        out_ref[:, :] = extract_topk(allc, K)

    return pl.pallas_call(
        body,
        out_shape=jax.ShapeDtypeStruct((m, K), jnp.float32),
        in_specs=[pl.BlockSpec(memory_space=pltpu.VMEM)],
        out_specs=pl.BlockSpec(memory_space=pltpu.VMEM),
        scratch_shapes=[
            pltpu.VMEM((m, N_CAND), jnp.float32),
            pltpu.VMEM((m, N_CAND), jnp.float32),
            pltpu.SemaphoreType.DMA,
            pltpu.SemaphoreType.DMA,
        ],
        compiler_params=pltpu.CompilerParams(collective_id=0),
    )(x)


# device time: 21099 ns/iter; 1.2174x vs baseline; 1.0141x over previous
import jax
import jax.numpy as jnp
from jax import lax
from jax.experimental import pallas as pl
from jax.experimental.pallas import tpu as pltpu

K = 16
NEG = float("-inf")
FOLD_LEVELS = 4
N_CAND = 81


def kernel(x):
    m, n = x.shape

    def extract_topk(a, k):
        if k == 1:
            return jnp.max(a, axis=1, keepdims=True)
        kcol = lax.broadcasted_iota(jnp.int32, (m, k), 1)
        vals = jnp.full((m, k), NEG, jnp.float32)
        for r in range(k):
            cur = jnp.max(a, axis=1, keepdims=True)
            vals = jnp.where(kcol == r, cur, vals)
            if r < k - 1:
                a = jnp.where(a == cur, NEG, a)
        return vals

    def extract_topk_batched(arrs, k):
        if k == 1:
            return [jnp.max(a, axis=1, keepdims=True) for a in arrs]
        b = jnp.stack(arrs, axis=0)
        c = len(arrs)
        kcol = lax.broadcasted_iota(jnp.int32, (c, m, k), 2)
        vals = jnp.full((c, m, k), NEG, jnp.float32)
        for r in range(k):
            cur = jnp.max(b, axis=2, keepdims=True)
            vals = jnp.where(kcol == r, cur, vals)
            if r < k - 1:
                b = jnp.where(b == cur, NEG, b)
        return [vals[i] for i in range(c)]

    def body(x_ref, out_ref, send_ref, recv_ref, send_sem, recv_sem):
        my_x = lax.axis_index("x")
        my_y = lax.axis_index("y")
        my_z = lax.axis_index("z")
        nbr = (my_x, 1 - my_y, my_z)

        jobs = [(x_ref[:, :], K)]
        for _ in range(FOLD_LEVELS):
            nxt = []
            for a, k in jobs:
                h = a.shape[1] // 2
                lo, hi = a[:, :h], a[:, h:]
                nxt.append((jnp.maximum(lo, hi), k))
                if k >= 2:
                    nxt.append((jnp.minimum(lo, hi), k // 2))
            jobs = nxt

        by_k: dict = {}
        for a, k in jobs:
            by_k.setdefault(k, []).append(a)
        pieces = []
        for k in sorted(by_k, reverse=True):
            pieces.extend(extract_topk_batched(by_k[k], k))
        cands = jnp.concatenate(pieces, axis=1)
        send_ref[:, :] = cands

        barrier_sem = pltpu.get_barrier_semaphore()
        pl.semaphore_signal(
            barrier_sem, inc=1, device_id=nbr,
            device_id_type=pl.DeviceIdType.MESH,
        )
        pl.semaphore_wait(barrier_sem, 1)

        rdma = pltpu.make_async_remote_copy(
            src_ref=send_ref,
            dst_ref=recv_ref,
            send_sem=send_sem,
            recv_sem=recv_sem,
            device_id=nbr,
            device_id_type=pl.DeviceIdType.MESH,
        )
        rdma.start()
        rdma.wait()

        allc = jnp.concatenate([cands, recv_ref[:, :]], axis=1)
        out_ref[:, :] = extract_topk(allc, K)

    return pl.pallas_call(
        body,
        out_shape=jax.ShapeDtypeStruct((m, K), jnp.float32),
        in_specs=[pl.BlockSpec(memory_space=pltpu.VMEM)],
        out_specs=pl.BlockSpec(memory_space=pltpu.VMEM),
        scratch_shapes=[
            pltpu.VMEM((m, N_CAND), jnp.float32),
            pltpu.VMEM((m, N_CAND), jnp.float32),
            pltpu.SemaphoreType.DMA,
            pltpu.SemaphoreType.DMA,
        ],
        compiler_params=pltpu.CompilerParams(collective_id=0),
    )(x)
